# baseline (speedup 1.0000x reference)
import functools

import jax
import jax.numpy as jnp
from jax import lax
from jax.experimental import pallas as pl
from jax.experimental.pallas import tpu as pltpu
from jax.experimental.pallas import tpu_sc as plsc

N = 10000
D = 128
E = 320000
EPS = 1e-5

NC = 2
NS = 16
NW = NC * NS
C = 125
K = 80
KH = K // 2
KH2 = KH // 2
ROWS_ACC = 10112
RPS = ROWS_ACC // NS

_mesh = plsc.VectorSubcoreMesh(core_axis_name="c", subcore_axis_name="s")


@functools.partial(
    pl.kernel,
    mesh=_mesh,
    out_type=jax.ShapeDtypeStruct((NC, ROWS_ACC, D), jnp.float32),
    scratch_types=[
        pltpu.VMEM((KH, C), jnp.int32),
        pltpu.VMEM((KH, C), jnp.int32),
        pltpu.VMEM((C, D), jnp.float32),
        pltpu.VMEM((C, D), jnp.float32),
        pltpu.VMEM_SHARED((ROWS_ACC, D), jnp.float32),
        pltpu.SemaphoreType.DMA,
        pltpu.SemaphoreType.DMA,
        pltpu.SemaphoreType.DMA,
        pltpu.SemaphoreType.DMA,
    ],
)
def _sc_scatter(ed_hbm, h_hbm, out_hbm,
                src_v, dst_v, buf_a, buf_b, acc, sem_ga, sem_gb, sem_sa, sem_sb):
    c = lax.axis_index("c")
    s = lax.axis_index("s")
    w = s * NC + c

    z16 = jnp.zeros((16,), jnp.float32)

    def zero_row(r, carry):
        for col in range(D // 16):
            buf_a[r, pl.ds(col * 16, 16)] = z16
        return carry

    lax.fori_loop(0, C, zero_row, 0, unroll=False)
    for t in range(5):
        pltpu.sync_copy(buf_a.at[pl.ds(0, 120)],
                        acc.at[pl.ds(s * RPS + t * 120, 120)])
    pltpu.sync_copy(buf_a.at[pl.ds(0, 32)], acc.at[pl.ds(s * RPS + 600, 32)])
    plsc.subcore_barrier()

    for p in range(2):
        pltpu.sync_copy(ed_hbm.at[0, w, pl.ds(p * KH, KH)], src_v)
        pltpu.sync_copy(ed_hbm.at[1, w, pl.ds(p * KH, KH)], dst_v)
        pltpu.async_copy(h_hbm.at[src_v.at[0]], buf_a, sem_ga)
        pltpu.async_copy(h_hbm.at[src_v.at[1]], buf_b, sem_gb)

        def body(i, carry):
            ja = 2 * i
            jb = 2 * i + 1
            pltpu.make_async_copy(h_hbm.at[src_v.at[ja]], buf_a, sem_ga).wait()
            pltpu.async_copy(buf_a, acc.at[dst_v.at[ja]], sem_sa, add=True)
            pltpu.make_async_copy(h_hbm.at[src_v.at[jb]], buf_b, sem_gb).wait()
            pltpu.async_copy(buf_b, acc.at[dst_v.at[jb]], sem_sb, add=True)
            pltpu.make_async_copy(buf_a, acc.at[dst_v.at[ja]], sem_sa).wait()
            pltpu.make_async_copy(buf_b, acc.at[dst_v.at[jb]], sem_sb).wait()

            @pl.when(i < KH2 - 1)
            def _():
                pltpu.async_copy(h_hbm.at[src_v.at[ja + 2]], buf_a, sem_ga)
                pltpu.async_copy(h_hbm.at[src_v.at[jb + 2]], buf_b, sem_gb)

            return carry

        lax.fori_loop(0, KH2, body, 0, unroll=False)
    plsc.subcore_barrier()
    pltpu.sync_copy(acc.at[pl.ds(s * RPS, RPS)],
                    out_hbm.at[c, pl.ds(s * RPS, RPS)])


def _tc_mm_bn(p_ref, w_ref, g_ref, b_ref, o_ref):
    agg = p_ref[0, :N, :] + p_ref[1, :N, :]
    z = lax.dot_general(agg, w_ref[...], (((1,), (1,)), ((), ())),
                        preferred_element_type=jnp.float32)
    mean = jnp.mean(z, axis=0, keepdims=True)
    zc = z - mean
    var = jnp.mean(zc * zc, axis=0, keepdims=True)
    o_ref[...] = zc * lax.rsqrt(var + EPS) * g_ref[...] + b_ref[...]


def kernel(h, edge_index, W, gamma, beta):
    ed = edge_index.astype(jnp.int32).reshape(2, NW, K, C)

    partial = _sc_scatter(ed, h)

    out = pl.pallas_call(
        _tc_mm_bn,
        out_shape=jax.ShapeDtypeStruct((N, D), jnp.float32),
    )(partial, W, gamma.reshape(1, D), beta.reshape(1, D))
    return out


# speedup vs baseline: 1.2762x; 1.2762x over previous
import functools

import jax
import jax.numpy as jnp
from jax import lax
from jax.experimental import pallas as pl
from jax.experimental.pallas import tpu as pltpu
from jax.experimental.pallas import tpu_sc as plsc

N = 10000
D = 128
E = 320000
EPS = 1e-5

NC = 2
NS = 16
NW = NC * NS
C = 125
K = 80
KH = K // 2
KH2 = KH // 2
ROWS_ACC = 10112
RPS = ROWS_ACC // NS

_mesh = plsc.VectorSubcoreMesh(core_axis_name="c", subcore_axis_name="s")


@functools.partial(
    pl.kernel,
    mesh=_mesh,
    out_type=jax.ShapeDtypeStruct((NC, ROWS_ACC, D), jnp.float32),
    scratch_types=[
        pltpu.VMEM((KH, C), jnp.int32),
        pltpu.VMEM((KH, C), jnp.int32),
        pltpu.VMEM((C, D), jnp.float32),
        pltpu.VMEM((C, D), jnp.float32),
        pltpu.VMEM_SHARED((ROWS_ACC, D), jnp.float32),
        pltpu.SemaphoreType.DMA,
        pltpu.SemaphoreType.DMA,
    ],
)
def _sc_scatter(ed_hbm, h_hbm, out_hbm,
                src_v, dst_v, buf_a, buf_b, acc, sem_ga, sem_gb):
    c = lax.axis_index("c")
    s = lax.axis_index("s")
    w = s * NC + c

    z16 = jnp.zeros((16,), jnp.float32)

    def zero_row(r, carry):
        for col in range(D // 16):
            buf_a[r, pl.ds(col * 16, 16)] = z16
        return carry

    lax.fori_loop(0, C, zero_row, 0, unroll=False)
    for t in range(5):
        pltpu.sync_copy(buf_a.at[pl.ds(0, 120)],
                        acc.at[pl.ds(s * RPS + t * 120, 120)])
    pltpu.sync_copy(buf_a.at[pl.ds(0, 32)], acc.at[pl.ds(s * RPS + 600, 32)])
    plsc.subcore_barrier()

    for p in range(2):
        pltpu.sync_copy(ed_hbm.at[0, w, pl.ds(p * KH, KH)], src_v)
        pltpu.sync_copy(ed_hbm.at[1, w, pl.ds(p * KH, KH)], dst_v)
        pltpu.async_copy(h_hbm.at[src_v.at[0]], buf_a, sem_ga)

        def body(i, carry):
            ja = 2 * i
            jb = 2 * i + 1
            pltpu.async_copy(h_hbm.at[src_v.at[jb]], buf_b, sem_gb)
            pltpu.make_async_copy(h_hbm.at[src_v.at[ja]], buf_a, sem_ga).wait()
            pltpu.sync_copy(buf_a, acc.at[dst_v.at[ja]], add=True)

            @pl.when(i < KH2 - 1)
            def _():
                pltpu.async_copy(h_hbm.at[src_v.at[ja + 2]], buf_a, sem_ga)

            pltpu.make_async_copy(h_hbm.at[src_v.at[jb]], buf_b, sem_gb).wait()
            pltpu.sync_copy(buf_b, acc.at[dst_v.at[jb]], add=True)
            return carry

        lax.fori_loop(0, KH2, body, 0, unroll=False)
    plsc.subcore_barrier()
    pltpu.sync_copy(acc.at[pl.ds(s * RPS, RPS)],
                    out_hbm.at[c, pl.ds(s * RPS, RPS)])


def _tc_mm_bn(p_ref, w_ref, g_ref, b_ref, o_ref):
    agg = p_ref[0, :N, :] + p_ref[1, :N, :]
    z = lax.dot_general(agg, w_ref[...], (((1,), (1,)), ((), ())),
                        preferred_element_type=jnp.float32)
    mean = jnp.mean(z, axis=0, keepdims=True)
    zc = z - mean
    var = jnp.mean(zc * zc, axis=0, keepdims=True)
    o_ref[...] = zc * lax.rsqrt(var + EPS) * g_ref[...] + b_ref[...]


def kernel(h, edge_index, W, gamma, beta):
    ed = edge_index.astype(jnp.int32).reshape(2, NW, K, C)

    partial = _sc_scatter(ed, h)

    out = pl.pallas_call(
        _tc_mm_bn,
        out_shape=jax.ShapeDtypeStruct((N, D), jnp.float32),
    )(partial, W, gamma.reshape(1, D), beta.reshape(1, D))
    return out
